# baseline (speedup 1.0000x reference)
import jax
import jax.numpy as jnp
from jax.experimental import pallas as pl
from jax.experimental.pallas import tpu as pltpu

_NE = 8192
_D = 32
_NT = 16384
_TILE = 256
_STEPS = _NT // _TILE
_CC = 0.25


def _vq_body(x_ref, w_ref, q_ref, cnt_ref, loss_ref, perp_ref, d_ref):
    step = pl.program_id(0)
    x = x_ref[...]
    w = w_ref[...]
    xsq = jnp.sum(x * x, axis=1, keepdims=True)
    wsq = jnp.sum(w * w, axis=1)
    mm = jax.lax.dot_general(
        x, w, dimension_numbers=(((1,), (1,)), ((), ())),
        preferred_element_type=jnp.float32)
    d_ref[...] = (xsq + wsq[None, :]) - 2.0 * mm
    d = d_ref[...]
    dmin = jnp.min(d, axis=1, keepdims=True)
    iota = jax.lax.broadcasted_iota(jnp.int32, (_TILE, _NE), 1)
    idx = jnp.min(jnp.where(d == dmin, iota, _NE), axis=1)
    onehot = (iota == idx[:, None]).astype(jnp.float32)
    q = jax.lax.dot_general(
        onehot, w, dimension_numbers=(((1,), (0,)), ((), ())),
        preferred_element_type=jnp.float32)
    q_ref[...] = x + (q - x)
    res = jnp.sum((q - x) * (q - x))
    cnt = jnp.sum(onehot, axis=0)

    @pl.when(step == 0)
    def _():
        cnt_ref[0, :] = cnt
        loss_ref[0, 0] = res

    @pl.when(step > 0)
    def _():
        cnt_ref[0, :] = cnt_ref[0, :] + cnt
        loss_ref[0, 0] = loss_ref[0, 0] + res

    @pl.when(step == _STEPS - 1)
    def _():
        m = loss_ref[0, 0] / (_NT * _D)
        loss_ref[0, 0] = m + _CC * m
        p = cnt_ref[0, :] * (1.0 / _NT)
        ent = jnp.sum(p * jnp.log(p + 1e-10))
        perp_ref[0, 0] = jnp.exp(-ent)


def kernel(inputs, embedding_weight):
    flat = inputs.reshape(-1, _D)
    q, _cnt, loss, perp = pl.pallas_call(
        _vq_body,
        grid=(_STEPS,),
        in_specs=[
            pl.BlockSpec((_TILE, _D), lambda i: (i, 0)),
            pl.BlockSpec((_NE, _D), lambda i: (0, 0)),
        ],
        out_specs=[
            pl.BlockSpec((_TILE, _D), lambda i: (i, 0)),
            pl.BlockSpec((1, _NE), lambda i: (0, 0)),
            pl.BlockSpec((1, 1), lambda i: (0, 0), memory_space=pltpu.SMEM),
            pl.BlockSpec((1, 1), lambda i: (0, 0), memory_space=pltpu.SMEM),
        ],
        out_shape=[
            jax.ShapeDtypeStruct((_NT, _D), jnp.float32),
            jax.ShapeDtypeStruct((1, _NE), jnp.float32),
            jax.ShapeDtypeStruct((1, 1), jnp.float32),
            jax.ShapeDtypeStruct((1, 1), jnp.float32),
        ],
        scratch_shapes=[pltpu.VMEM((_TILE, _NE), jnp.float32)],
        compiler_params=pltpu.CompilerParams(
            dimension_semantics=("arbitrary",)),
    )(flat, embedding_weight)
    return q.reshape(inputs.shape), loss[0, 0], perp[0, 0]
